# baseline (speedup 1.0000x reference)
import functools

import jax
import jax.numpy as jnp
from jax.experimental import pallas as pl
from jax.experimental.pallas import tpu as pltpu

M_BLK = 32
K_CHUNK = 6272


def _mm_kernel(c_ref, emit_ref, pos_ref, out_ref, *, k_total):
    acc = jnp.broadcast_to(pos_ref[...], out_ref.shape).astype(jnp.float32)
    for start in range(0, k_total, K_CHUNK):
        width = min(K_CHUNK, k_total - start)
        x = c_ref[:, start:start + width].astype(jnp.bfloat16)
        e = emit_ref[start:start + width, :]
        acc += jnp.dot(x, e, preferred_element_type=jnp.float32)
    out_ref[...] = acc


def kernel(choices, chunk_idx, float_emit, pos_embed):
    pos_row = jax.lax.dynamic_slice_in_dim(pos_embed, chunk_idx, 1, axis=0)
    n, k_total = choices.shape
    chunk_dim = float_emit.shape[1]
    emit_bf = float_emit.astype(jnp.bfloat16)

    return pl.pallas_call(
        functools.partial(_mm_kernel, k_total=k_total),
        grid=(n // M_BLK,),
        in_specs=[
            pl.BlockSpec((M_BLK, k_total), lambda m: (m, 0)),
            pl.BlockSpec((k_total, chunk_dim), lambda m: (0, 0)),
            pl.BlockSpec((1, chunk_dim), lambda m: (0, 0)),
        ],
        out_specs=pl.BlockSpec((M_BLK, chunk_dim), lambda m: (m, 0)),
        out_shape=jax.ShapeDtypeStruct((n, chunk_dim), jnp.float32),
        compiler_params=pltpu.CompilerParams(
            dimension_semantics=("arbitrary",),
        ),
    )(choices, emit_bf, pos_row)


# speedup vs baseline: 2.5299x; 2.5299x over previous
import functools

import jax
import jax.numpy as jnp
from jax.experimental import pallas as pl
from jax.experimental.pallas import tpu as pltpu

M_BLK = 128
K_CHUNK = 6272


def _mm_kernel(c8_ref, emit_ref, pos_ref, out_ref, *, k_total):
    acc = jnp.broadcast_to(pos_ref[...], out_ref.shape).astype(jnp.float32)
    for start in range(0, k_total, K_CHUNK):
        width = min(K_CHUNK, k_total - start)
        x = c8_ref[:, start:start + width].astype(jnp.bfloat16)
        e = emit_ref[start:start + width, :]
        acc += jnp.dot(x, e, preferred_element_type=jnp.float32)
    out_ref[...] = acc


def kernel(choices, chunk_idx, float_emit, pos_embed):
    pos_row = jax.lax.dynamic_slice_in_dim(pos_embed, chunk_idx, 1, axis=0)
    n, k_total = choices.shape
    chunk_dim = float_emit.shape[1]
    emit_bf = float_emit.astype(jnp.bfloat16)
    c8 = choices.astype(jnp.int8)

    return pl.pallas_call(
        functools.partial(_mm_kernel, k_total=k_total),
        grid=(n // M_BLK,),
        in_specs=[
            pl.BlockSpec((M_BLK, k_total), lambda m: (m, 0)),
            pl.BlockSpec((k_total, chunk_dim), lambda m: (0, 0)),
            pl.BlockSpec((1, chunk_dim), lambda m: (0, 0)),
        ],
        out_specs=pl.BlockSpec((M_BLK, chunk_dim), lambda m: (m, 0)),
        out_shape=jax.ShapeDtypeStruct((n, chunk_dim), jnp.float32),
        compiler_params=pltpu.CompilerParams(
            dimension_semantics=("arbitrary",),
        ),
    )(c8, emit_bf, pos_row)


# speedup vs baseline: 2.5381x; 1.0033x over previous
import functools

import jax
import jax.numpy as jnp
from jax.experimental import pallas as pl
from jax.experimental.pallas import tpu as pltpu

M_BLK = 128
K_CHUNK = 6272


def _mm_kernel(c8_ref, emit_ref, pos_ref, out_ref, *, k_total):
    acc = jnp.broadcast_to(pos_ref[...], out_ref.shape).astype(jnp.float32)
    for start in range(0, k_total, K_CHUNK):
        width = min(K_CHUNK, k_total - start)
        x = c8_ref[:, start:start + width].astype(jnp.bfloat16)
        e = emit_ref[start:start + width, :]
        acc += jnp.dot(x, e, preferred_element_type=jnp.float32)
    out_ref[...] = acc


def kernel(choices, chunk_idx, float_emit, pos_embed):
    pos_row = jax.lax.dynamic_slice_in_dim(pos_embed, chunk_idx, 1, axis=0)
    n, k_total = choices.shape
    chunk_dim = float_emit.shape[1]
    emit_bf = float_emit.astype(jnp.bfloat16)
    c8 = jnp.where(choices, jnp.int8(1), jnp.int8(0))

    return pl.pallas_call(
        functools.partial(_mm_kernel, k_total=k_total),
        grid=(n // M_BLK,),
        in_specs=[
            pl.BlockSpec((M_BLK, k_total), lambda m: (m, 0)),
            pl.BlockSpec((k_total, chunk_dim), lambda m: (0, 0)),
            pl.BlockSpec((1, chunk_dim), lambda m: (0, 0)),
        ],
        out_specs=pl.BlockSpec((M_BLK, chunk_dim), lambda m: (m, 0)),
        out_shape=jax.ShapeDtypeStruct((n, chunk_dim), jnp.float32),
        compiler_params=pltpu.CompilerParams(
            dimension_semantics=("arbitrary",),
        ),
    )(c8, emit_bf, pos_row)
